# Initial kernel scaffold; baseline (speedup 1.0000x reference)
import jax
import jax.numpy as jnp
from jax.experimental import pallas as pl


def kernel(pred):
    raise NotImplementedError("write your pallas kernel here")


# speedup vs baseline: 60.9802x; 60.9802x over previous
import functools

import jax
import jax.numpy as jnp
from jax import lax
from jax.experimental import pallas as pl
from jax.experimental.pallas import tpu as pltpu
from jax.experimental.pallas import tpu_sc as plsc

NB = 128
NG = NB // 16
NCLS = 20
IOU_T = 0.4
SCORE_T = 0.05
MAXOUT = 100


def _argmax_slots(ms, lane):
    best = jnp.max(ms[0])
    for g in range(1, NG):
        best = jnp.maximum(best, jnp.max(ms[g]))
    pos = jnp.int32(2 * NB)
    for g in range(NG):
        cand = jnp.where(ms[g] == best, lane + g * 16, 2 * NB)
        pos = jnp.minimum(pos, jnp.min(cand))
    return best, pos


def _sc_body(pred_hbm, nv_hbm, box_hbm, sc_hbm, cls_hbm,
             pred_v, y1v, x1v, y2v, x2v, sv, lsc, lbx, lcl,
             box_st, sc_st, cls_st, nv_st):
    info = plsc.get_sparse_core_info()
    nw = info.num_cores * info.num_subcores
    ipw = 64 // nw
    wid = lax.axis_index("s") * info.num_cores + lax.axis_index("c")
    lane = lax.iota(jnp.int32, 16)
    zf = jnp.zeros((16,), jnp.float32)
    zi = jnp.zeros((16,), jnp.int32)
    m0 = lane == 0

    def sstore(ref, idx, val):
        plsc.store_scatter(ref, [zi + idx], jnp.zeros((16,), val.dtype) + val,
                           mask=m0)

    def sload(ref, idx):
        return plsc.load_gather(ref, [zi + idx])[0]

    for t in range(ipw):
        img = wid * ipw + t
        pltpu.sync_copy(pred_hbm.at[img], pred_v)

        for g in range(4):
            cell = lane + g * 16
            ii = (cell // 7).astype(jnp.float32)
            jj = (cell % 7).astype(jnp.float32)
            p = [pred_v[pl.ds(k * 64 + g * 16, 16)] for k in range(30)]
            maxv = p[10]
            for c in range(1, NCLS):
                maxv = jnp.maximum(maxv, p[10 + c])
            for off, base in ((0, 0), (5, 64)):
                cx = ii * 64.0 + p[off] * 64.0
                cy = jj * 64.0 + p[off + 1] * 64.0
                w = jnp.minimum(p[off + 2] * 448.0, 448.0)
                h = jnp.minimum(p[off + 3] * 448.0, 448.0)
                sl = pl.ds(base + g * 16, 16)
                y1v[sl] = cy - h / 2.0
                x1v[sl] = cx - w / 2.0
                y2v[sl] = cy + h / 2.0
                x2v[sl] = cx + w / 2.0
                conf = p[off + 4]
                for c in range(NCLS):
                    v = p[10 + c]
                    s = jnp.where(v == maxv, v * conf, 0.0)
                    sv[pl.ds(c * NB + base + g * 16, 16)] = jnp.where(
                        s > SCORE_T, s, -1.0)

        neg1 = zf - 1.0
        for g in range(NG):
            lsc[pl.ds(g * 16, 16)] = neg1

        def class_body(c, K):
            ms0 = [sv[pl.ds(c * NB + g * 16, 16)] for g in range(NG)]
            best0, p0 = _argmax_slots(ms0, lane)

            def cond(carry):
                return carry[NG] > 0.0

            def body(carry):
                ms = list(carry[:NG])
                best, pos, k = carry[NG], carry[NG + 1], carry[NG + 2]
                kc = jnp.minimum(k, NB - 1)
                sstore(lsc, kc, best)
                sstore(lbx, kc, pos)
                sstore(lcl, kc, c.astype(jnp.float32))
                yb1 = sload(y1v, pos)
                xb1 = sload(x1v, pos)
                yb2 = sload(y2v, pos)
                xb2 = sload(x2v, pos)
                areab = (yb2 - yb1) * (xb2 - xb1)
                nms = []
                for g in range(NG):
                    sl = pl.ds(g * 16, 16)
                    ay1 = y1v[sl]
                    ax1 = x1v[sl]
                    ay2 = y2v[sl]
                    ax2 = x2v[sl]
                    ih = jnp.maximum(jnp.minimum(yb2, ay2) - jnp.maximum(yb1, ay1), 0.0)
                    iw = jnp.maximum(jnp.minimum(xb2, ax2) - jnp.maximum(xb1, ax1), 0.0)
                    inter = ih * iw
                    union = areab + (ay2 - ay1) * (ax2 - ax1) - inter
                    iou = jnp.where(union > 0.0, inter / union, 0.0)
                    nms.append(jnp.where(iou > IOU_T, -1.0, ms[g]))
                nbest, npos = _argmax_slots(nms, lane)
                return (*nms, nbest, npos, k + 1)

            out = lax.while_loop(cond, body, (*ms0, best0, p0, K))
            return out[NG + 2]

        K = lax.fori_loop(0, NCLS, class_body, jnp.int32(0))

        zi = jnp.zeros((16,), jnp.int32)
        for g in range(25):
            box_st[pl.ds(g * 16, 16)] = zf
        for g in range(NG):
            sc_st[pl.ds(g * 16, 16)] = zf
            cls_st[pl.ds(g * 16, 16)] = zf
        kv = jnp.minimum(K, MAXOUT)

        ls0 = [lsc[pl.ds(g * 16, 16)] for g in range(NG)]

        def sel_cond(carry):
            return carry[NG] < kv

        def sel_body(carry):
            ls = list(carry[:NG])
            e = carry[NG]
            best, pos = _argmax_slots(ls, lane)
            sstore(sc_st, e, best)
            sstore(cls_st, e, sload(lcl, pos))
            bi = sload(lbx, pos)
            vals = jnp.where(lane == 0, sload(y1v, bi),
                             jnp.where(lane == 1, sload(x1v, bi),
                                       jnp.where(lane == 2, sload(y2v, bi),
                                                 sload(x2v, bi))))
            plsc.store_scatter(box_st, [zi + e * 4 + lane], vals, mask=lane < 4)
            nls = [jnp.where(lane + g * 16 == pos, -1.0, ls[g]) for g in range(NG)]
            return (*nls, e + 1)

        lax.while_loop(sel_cond, sel_body, (*ls0, jnp.int32(0)))

        nv_st[pl.ds(0, 16)] = zi + kv

        pltpu.sync_copy(nv_st, nv_hbm.at[img])
        pltpu.sync_copy(box_st, box_hbm.at[img])
        pltpu.sync_copy(sc_st, sc_hbm.at[img])
        pltpu.sync_copy(cls_st, cls_hbm.at[img])


def kernel(pred):
    B = pred.shape[0]
    pt = jnp.transpose(pred.reshape(B, 49, 30), (0, 2, 1))
    pt = jnp.pad(pt, ((0, 0), (0, 0), (0, 15)))
    pflat = pt.reshape(B, 30 * 64)

    mesh = plsc.VectorSubcoreMesh(core_axis_name="c", subcore_axis_name="s")
    f = pl.kernel(
        _sc_body,
        out_type=[
            jax.ShapeDtypeStruct((B, 16), jnp.int32),
            jax.ShapeDtypeStruct((B, 400), jnp.float32),
            jax.ShapeDtypeStruct((B, NB), jnp.float32),
            jax.ShapeDtypeStruct((B, NB), jnp.float32),
        ],
        mesh=mesh,
        compiler_params=pltpu.CompilerParams(needs_layout_passes=False),
        scratch_types=[
            pltpu.VMEM((30 * 64,), jnp.float32),
            pltpu.VMEM((NB,), jnp.float32),
            pltpu.VMEM((NB,), jnp.float32),
            pltpu.VMEM((NB,), jnp.float32),
            pltpu.VMEM((NB,), jnp.float32),
            pltpu.VMEM((NCLS * NB,), jnp.float32),
            pltpu.VMEM((NB,), jnp.float32),
            pltpu.VMEM((NB,), jnp.int32),
            pltpu.VMEM((NB,), jnp.float32),
            pltpu.VMEM((400,), jnp.float32),
            pltpu.VMEM((NB,), jnp.float32),
            pltpu.VMEM((NB,), jnp.float32),
            pltpu.VMEM((16,), jnp.int32),
        ],
    )
    nv, boxes, sc, cls = f(pflat)
    return (nv[:, 0], boxes.reshape(B, 100, 4), sc[:, :MAXOUT], cls[:, :MAXOUT])


# speedup vs baseline: 70.7828x; 1.1607x over previous
import functools

import jax
import jax.numpy as jnp
from jax import lax
from jax.experimental import pallas as pl
from jax.experimental.pallas import tpu as pltpu
from jax.experimental.pallas import tpu_sc as plsc

NB = 128
NG = NB // 16
NCLS = 20
IOU_T = 0.4
SCORE_T = 0.05
MAXOUT = 100
REC = 784
O_BOX = 0
O_SC = 400
O_CLS = 528
O_NV = 656


def _argmax_slots(ms, lane):
    m = ms[0]
    for g in range(1, NG):
        m = jnp.maximum(m, ms[g])
    best = jnp.max(m)
    pv = jnp.where(ms[0] == best, lane, 2 * NB)
    for g in range(1, NG):
        pv = jnp.minimum(pv, jnp.where(ms[g] == best, lane + g * 16, 2 * NB))
    pos = jnp.min(pv)
    return best, pos


def _sc_body(pred_hbm, out_hbm,
             pred_v0, pred_v1, y1v, x1v, y2v, x2v, sv, lsl, lbx, ost,
             sem0, sem1):
    info = plsc.get_sparse_core_info()
    nw = info.num_cores * info.num_subcores
    ipw = 64 // nw
    wid = lax.axis_index("s") * info.num_cores + lax.axis_index("c")
    lane = lax.iota(jnp.int32, 16)
    zf = jnp.zeros((16,), jnp.float32)
    zi = jnp.zeros((16,), jnp.int32)
    m0 = lane == 0

    preds = (pred_v0, pred_v1)
    cps = [pltpu.async_copy(pred_hbm.at[wid * ipw + t], preds[t], sem)
           for t, sem in zip(range(ipw), (sem0, sem1))]

    for g in range(ipw * REC // 16):
        ost[pl.ds(g * 16, 16)] = zf

    for t in range(ipw):
        cps[t].wait()
        pred_v = preds[t]
        base = t * REC

        for g in range(4):
            cell = lane + g * 16
            ii = (cell // 7).astype(jnp.float32)
            jj = (cell % 7).astype(jnp.float32)
            p = [pred_v[pl.ds(k * 64 + g * 16, 16)] for k in range(30)]
            maxv = p[10]
            for c in range(1, NCLS):
                maxv = jnp.maximum(maxv, p[10 + c])
            for off, sbase in ((0, 0), (5, 64)):
                cx = ii * 64.0 + p[off] * 64.0
                cy = jj * 64.0 + p[off + 1] * 64.0
                w = jnp.minimum(p[off + 2] * 448.0, 448.0)
                h = jnp.minimum(p[off + 3] * 448.0, 448.0)
                sl = pl.ds(sbase + g * 16, 16)
                y1v[sl] = cy - h / 2.0
                x1v[sl] = cx - w / 2.0
                y2v[sl] = cy + h / 2.0
                x2v[sl] = cx + w / 2.0
                conf = p[off + 4]
                for c in range(NCLS):
                    v = p[10 + c]
                    s = jnp.where(v == maxv, v * conf, 0.0)
                    sv[pl.ds(c * NB + sbase + g * 16, 16)] = jnp.where(
                        s > SCORE_T, s, -1.0)

        neg1 = zf - 1.0
        for g in range(NG):
            lsl[pl.ds(g * 16, 16)] = neg1

        def class_body(c, K):
            ms0 = [sv[pl.ds(c * NB + g * 16, 16)] for g in range(NG)]
            best0, p0 = _argmax_slots(ms0, lane)

            def cond(carry):
                return carry[NG] > 0.0

            def body(carry):
                ms = list(carry[:NG])
                best, pos, k = carry[NG], carry[NG + 1], carry[NG + 2]
                kc = jnp.minimum(k, NB - 1)
                av = jnp.where(m0, zf + best, zf + c.astype(jnp.float32))
                ai = jnp.where(m0, zi + kc, zi + NB + kc)
                plsc.store_scatter(lsl, [ai], av, mask=lane < 2)
                plsc.store_scatter(lbx, [zi + kc], zi + pos, mask=m0)
                pv = zi + pos
                yb1 = plsc.load_gather(y1v, [pv])
                xb1 = plsc.load_gather(x1v, [pv])
                yb2 = plsc.load_gather(y2v, [pv])
                xb2 = plsc.load_gather(x2v, [pv])
                areab = (yb2 - yb1) * (xb2 - xb1)
                nms = []
                for g in range(NG):
                    sl = pl.ds(g * 16, 16)
                    ay1 = y1v[sl]
                    ax1 = x1v[sl]
                    ay2 = y2v[sl]
                    ax2 = x2v[sl]
                    ih = jnp.maximum(jnp.minimum(yb2, ay2) - jnp.maximum(yb1, ay1), 0.0)
                    iw = jnp.maximum(jnp.minimum(xb2, ax2) - jnp.maximum(xb1, ax1), 0.0)
                    inter = ih * iw
                    union = areab + (ay2 - ay1) * (ax2 - ax1) - inter
                    iou = jnp.where(union > 0.0, inter / union, 0.0)
                    nms.append(jnp.where(iou > IOU_T, -1.0, ms[g]))
                nbest, npos = _argmax_slots(nms, lane)
                return (*nms, nbest, npos, k + 1)

            out = lax.while_loop(cond, body, (*ms0, best0, p0, K))
            return out[NG + 2]

        K = lax.fori_loop(0, NCLS, class_body, jnp.int32(0))

        kv = jnp.minimum(K, MAXOUT)
        ls0 = [lsl[pl.ds(g * 16, 16)] for g in range(NG)]

        def sel_cond(carry):
            return carry[NG] < kv

        def sel_body(carry):
            ls = list(carry[:NG])
            e = carry[NG]
            best, pos = _argmax_slots(ls, lane)
            pv = zi + pos
            clsv = plsc.load_gather(lsl, [pv + NB])
            biv = plsc.load_gather(lbx, [pv])
            y1b = plsc.load_gather(y1v, [biv])
            x1b = plsc.load_gather(x1v, [biv])
            y2b = plsc.load_gather(y2v, [biv])
            x2b = plsc.load_gather(x2v, [biv])
            bv = jnp.where(lane == 0, y1b,
                           jnp.where(lane == 1, x1b,
                                     jnp.where(lane == 2, y2b, x2b)))
            plsc.store_scatter(ost, [zi + (base + O_BOX) + e * 4 + lane], bv,
                               mask=lane < 4)
            av = jnp.where(m0, zf + best, clsv)
            ai = jnp.where(m0, zi + (base + O_SC) + e, zi + (base + O_CLS) + e)
            plsc.store_scatter(ost, [ai], av, mask=lane < 2)
            nls = [jnp.where(lane + g * 16 == pos, -1.0, ls[g]) for g in range(NG)]
            return (*nls, e + 1)

        lax.while_loop(sel_cond, sel_body, (*ls0, jnp.int32(0)))

        plsc.store_scatter(ost, [zi + (base + O_NV)],
                           zf + kv.astype(jnp.float32), mask=m0)

    pltpu.sync_copy(ost, out_hbm.at[wid])


def kernel(pred):
    B = pred.shape[0]
    pt = jnp.transpose(pred.reshape(B, 49, 30), (0, 2, 1))
    pt = jnp.pad(pt, ((0, 0), (0, 0), (0, 15)))
    pflat = pt.reshape(B, 30 * 64)

    mesh = plsc.VectorSubcoreMesh(core_axis_name="c", subcore_axis_name="s")
    f = pl.kernel(
        _sc_body,
        out_type=[jax.ShapeDtypeStruct((32, 2 * REC), jnp.float32)],
        mesh=mesh,
        compiler_params=pltpu.CompilerParams(needs_layout_passes=False),
        scratch_types=[
            pltpu.VMEM((30 * 64,), jnp.float32),
            pltpu.VMEM((30 * 64,), jnp.float32),
            pltpu.VMEM((NB,), jnp.float32),
            pltpu.VMEM((NB,), jnp.float32),
            pltpu.VMEM((NB,), jnp.float32),
            pltpu.VMEM((NB,), jnp.float32),
            pltpu.VMEM((NCLS * NB,), jnp.float32),
            pltpu.VMEM((2 * NB,), jnp.float32),
            pltpu.VMEM((NB,), jnp.int32),
            pltpu.VMEM((2 * REC,), jnp.float32),
            pltpu.SemaphoreType.DMA,
            pltpu.SemaphoreType.DMA,
        ],
    )
    o = f(pflat)
    if isinstance(o, (tuple, list)):
        o = o[0]
    o = o.reshape(B, REC)
    boxes = o[:, O_BOX:O_BOX + 400].reshape(B, MAXOUT, 4)
    sc = o[:, O_SC:O_SC + MAXOUT]
    cls = o[:, O_CLS:O_CLS + MAXOUT]
    nv = o[:, O_NV].astype(jnp.int32)
    return (nv, boxes, sc, cls)
